# Initial kernel scaffold; baseline (speedup 1.0000x reference)
import jax
import jax.numpy as jnp
from jax import lax
from jax.experimental import pallas as pl
from jax.experimental.pallas import tpu as pltpu
from jax.experimental.pallas import tpu_sc as plsc

N = 10000
E = 320000
D_H = 128
G = 64

NC = 2
NS = 16
NW = NC * NS
K = 128
EN = E + N
CHUNKS = -(-EN // (NW * K))
PER_TILE = CHUNKS * K
EN_PAD = PER_TILE * NW
NPAD = -(-N // NS) // 16 * 16 * NS + (0 if N % NS == 0 else 0)
NPAD = ((N + NS * 16 - 1) // (NS * 16)) * NS * 16
ROWS_OUT = NPAD // NS
BLK = 1000
NBLK = N // BLK

_MESH = plsc.VectorSubcoreMesh(core_axis_name="c", subcore_axis_name="s",
                               num_cores=NC, num_subcores=NS)


def _sc_deg_body(col_hbm, out_hbm, acc, ones_v, cidx):
    c = lax.axis_index("c")
    s = lax.axis_index("s")
    base_r = s * ROWS_OUT

    @pl.loop(0, K)
    def _(r):
        ones_v[r, :] = jnp.zeros((16,), jnp.float32)

    for t in range(ROWS_OUT // K):
        pltpu.sync_copy(ones_v, acc.at[pl.ds(base_r + t * K, K)])
    rem = ROWS_OUT % K
    if rem:
        pltpu.sync_copy(ones_v.at[pl.ds(0, rem)],
                        acc.at[pl.ds(base_r + (ROWS_OUT // K) * K, rem)])

    @pl.loop(0, K)
    def _(r):
        ones_v[r, :] = jnp.ones((16,), jnp.float32)

    plsc.subcore_barrier()

    base_e = c * (EN_PAD // NC) + s * PER_TILE

    @pl.loop(0, CHUNKS)
    def _(i):
        off = base_e + i * K
        pltpu.sync_copy(col_hbm.at[pl.ds(off, K)], cidx)
        pltpu.sync_copy(ones_v, acc.at[cidx], add=True)

    plsc.subcore_barrier()
    pltpu.sync_copy(acc.at[pl.ds(base_r, ROWS_OUT)],
                    out_hbm.at[c].at[pl.ds(base_r, ROWS_OUT)])


def _sc_deg(col_p):
    kern = pl.kernel(
        _sc_deg_body,
        out_type=jax.ShapeDtypeStruct((NC, NPAD, 16), jnp.float32),
        mesh=_MESH,
        scratch_types=[
            pltpu.VMEM_SHARED((NPAD, 16), jnp.float32),
            pltpu.VMEM((K, 16), jnp.float32),
            pltpu.VMEM((K,), jnp.int32),
        ],
    )
    return kern(col_p)


def _sc_agg_body(v_hbm, row_hbm, col_hbm, nq_hbm, out_hbm,
                 acc, rows_v, ridx, cidx, nqv):
    c = lax.axis_index("c")
    s = lax.axis_index("s")
    base_r = s * ROWS_OUT

    @pl.loop(0, K)
    def _(r):
        for g in range(8):
            rows_v[r, pl.ds(g * 16, 16)] = jnp.zeros((16,), jnp.float32)

    for t in range(ROWS_OUT // K):
        pltpu.sync_copy(rows_v, acc.at[pl.ds(base_r + t * K, K)])
    rem = ROWS_OUT % K
    if rem:
        pltpu.sync_copy(rows_v.at[pl.ds(0, rem)],
                        acc.at[pl.ds(base_r + (ROWS_OUT // K) * K, rem)])

    plsc.subcore_barrier()

    base_e = c * (EN_PAD // NC) + s * PER_TILE

    @pl.loop(0, CHUNKS)
    def _(i):
        off = base_e + i * K
        pltpu.sync_copy(row_hbm.at[pl.ds(off, K)], ridx)
        pltpu.sync_copy(col_hbm.at[pl.ds(off, K)], cidx)
        pltpu.sync_copy(nq_hbm.at[pl.ds(off, K)], nqv)
        pltpu.sync_copy(v_hbm.at[ridx], rows_v)

        @pl.loop(0, K)
        def _(k):
            lanes = jnp.zeros((16,), jnp.int32) + k
            nqk = plsc.load_gather(nqv, [lanes])
            for g in range(8):
                sl = pl.ds(g * 16, 16)
                rows_v[k, sl] = rows_v[k, sl] * nqk

        pltpu.sync_copy(rows_v, acc.at[cidx], add=True)

    plsc.subcore_barrier()
    pltpu.sync_copy(acc.at[pl.ds(base_r, ROWS_OUT)],
                    out_hbm.at[c].at[pl.ds(base_r, ROWS_OUT)])


def _sc_agg(v, row_p, col_p, nq_p):
    kern = pl.kernel(
        _sc_agg_body,
        out_type=jax.ShapeDtypeStruct((NC, NPAD, D_H), jnp.float32),
        mesh=_MESH,
        scratch_types=[
            pltpu.VMEM_SHARED((NPAD, D_H), jnp.float32),
            pltpu.VMEM((K, D_H), jnp.float32),
            pltpu.VMEM((K,), jnp.int32),
            pltpu.VMEM((K,), jnp.int32),
            pltpu.VMEM((K,), jnp.float32),
        ],
    )
    return kern(v, row_p, col_p, nq_p)


def _dis_from(deg_ref):
    d = deg_ref[0] + deg_ref[1]
    return lax.rsqrt(d[:, 0:1])


def _tc_layer1_body(x_ref, w_ref, deg_ref, o_ref):
    dis = _dis_from(deg_ref)
    o_ref[...] = jnp.dot(x_ref[...], w_ref[...],
                         preferred_element_type=jnp.float32) * dis


def _tc_layer1(x, W, degs):
    return pl.pallas_call(
        _tc_layer1_body,
        grid=(NBLK,),
        in_specs=[
            pl.BlockSpec((BLK, D_H), lambda i: (i, 0)),
            pl.BlockSpec((D_H, D_H), lambda i: (0, 0)),
            pl.BlockSpec((NC, BLK, 16), lambda i: (0, i, 0)),
        ],
        out_specs=pl.BlockSpec((BLK, D_H), lambda i: (i, 0)),
        out_shape=jax.ShapeDtypeStruct((N, D_H), jnp.float32),
    )(x, W, degs)


def _tc_layer_body(g_ref, deg_ref, b_ref, w_ref, o_ref):
    dis = _dis_from(deg_ref)
    h = jnp.maximum((g_ref[0] + g_ref[1]) * dis + b_ref[...], 0.0)
    o_ref[...] = jnp.dot(h, w_ref[...],
                         preferred_element_type=jnp.float32) * dis


def _tc_layer(g, degs, b, W):
    return pl.pallas_call(
        _tc_layer_body,
        grid=(NBLK,),
        in_specs=[
            pl.BlockSpec((NC, BLK, D_H), lambda i: (0, i, 0)),
            pl.BlockSpec((NC, BLK, 16), lambda i: (0, i, 0)),
            pl.BlockSpec((1, D_H), lambda i: (0, 0)),
            pl.BlockSpec((D_H, D_H), lambda i: (0, 0)),
        ],
        out_specs=pl.BlockSpec((BLK, D_H), lambda i: (i, 0)),
        out_shape=jax.ShapeDtypeStruct((N, D_H), jnp.float32),
    )(g, degs, b.reshape(1, D_H), W)


def _tc_final_body(g_ref, deg_ref, b_ref, batch_ref, wc_ref, bc_ref, o_ref,
                   sums_scr, cnt_scr):
    i = pl.program_id(0)

    @pl.when(i == 0)
    def _():
        sums_scr[...] = jnp.zeros_like(sums_scr)
        cnt_scr[...] = jnp.zeros_like(cnt_scr)

    dis = _dis_from(deg_ref)
    h = jnp.maximum((g_ref[0] + g_ref[1]) * dis + b_ref[...], 0.0)
    onehot = (batch_ref[...] ==
              lax.broadcasted_iota(jnp.int32, (1, G), 1)).astype(jnp.float32)
    sums_scr[...] += lax.dot_general(onehot, h, (((0,), (0,)), ((), ())),
                                     preferred_element_type=jnp.float32)
    cnt_scr[...] += lax.dot_general(onehot, jnp.ones((BLK, D_H), jnp.float32),
                                    (((0,), (0,)), ((), ())),
                                    preferred_element_type=jnp.float32)

    @pl.when(i == pl.num_programs(0) - 1)
    def _():
        pooled = sums_scr[...] / jnp.maximum(cnt_scr[...], 1.0)
        o_ref[...] = jnp.dot(pooled, wc_ref[...],
                             preferred_element_type=jnp.float32) + bc_ref[...]


def _tc_final(g, degs, b, batch2, Wc, bc):
    return pl.pallas_call(
        _tc_final_body,
        grid=(NBLK,),
        in_specs=[
            pl.BlockSpec((NC, BLK, D_H), lambda i: (0, i, 0)),
            pl.BlockSpec((NC, BLK, 16), lambda i: (0, i, 0)),
            pl.BlockSpec((1, D_H), lambda i: (0, 0)),
            pl.BlockSpec((BLK, 1), lambda i: (i, 0)),
            pl.BlockSpec((D_H, G), lambda i: (0, 0)),
            pl.BlockSpec((1, G), lambda i: (0, 0)),
        ],
        out_specs=pl.BlockSpec((G, G), lambda i: (0, 0)),
        out_shape=jax.ShapeDtypeStruct((G, G), jnp.float32),
        scratch_shapes=[
            pltpu.VMEM((G, D_H), jnp.float32),
            pltpu.VMEM((G, D_H), jnp.float32),
        ],
    )(g, degs, b.reshape(1, D_H), batch2, Wc, bc.reshape(1, G))


def kernel(x, edge_index, batch, node_centrality, edge_centrality,
           W1, b1, W2, b2, W3, b3, Wc, bc):
    loops = jnp.arange(N, dtype=jnp.int32)
    row = jnp.concatenate([edge_index[0], loops])
    col = jnp.concatenate([edge_index[1], loops])
    nq = jnp.concatenate([edge_centrality, node_centrality])

    pad = EN_PAD - EN
    row_p = jnp.concatenate([row, jnp.zeros((pad,), jnp.int32)])
    col_p = jnp.concatenate([col, jnp.full((pad,), N, jnp.int32)])
    nq_p = jnp.concatenate([nq, jnp.zeros((pad,), jnp.float32)])

    deg_p = _sc_deg(col_p)
    degs = deg_p[:, :N, :]

    v = _tc_layer1(x, W1, degs)
    g = _sc_agg(v, row_p, col_p, nq_p)
    v = _tc_layer(g[:, :N, :], degs, b1, W2)
    g = _sc_agg(v, row_p, col_p, nq_p)
    v = _tc_layer(g[:, :N, :], degs, b2, W3)
    g = _sc_agg(v, row_p, col_p, nq_p)

    batch2 = batch.reshape(N, 1)
    return _tc_final(g[:, :N, :], degs, b3, batch2, Wc, bc)


# speedup vs baseline: 8.1030x; 8.1030x over previous
import dataclasses

import jax
import jax.numpy as jnp
from jax import lax
from jax.experimental import pallas as pl
from jax.experimental.pallas import tpu as pltpu
from jax.experimental.pallas import tpu_sc as plsc

N = 10000
E = 320000
D_H = 128
G = 64

NC = 2
NS = 16
NW = NC * NS
K = 128
EN = E + N
CHUNKS = -(-EN // (NW * K))
PER_TILE = CHUNKS * K
EN_PAD = PER_TILE * NW
NPAD = ((N + 1 + NS * 8 - 1) // (NS * 8)) * NS * 8
ROWS_OUT = NPAD // NS
BLK = 1000
NBLK = N // BLK

_MESH = plsc.VectorSubcoreMesh(core_axis_name="c", subcore_axis_name="s",
                               num_cores=NC, num_subcores=NS)

_SC_PARAMS = pltpu.CompilerParams()
if "needs_layout_passes" in pltpu.CompilerParams.__dataclass_fields__:
    _SC_PARAMS = dataclasses.replace(_SC_PARAMS, needs_layout_passes=False)


def _sc_deg_body(col_hbm, out_hbm, acc, ones_v, cidx):
    c = lax.axis_index("c")
    s = lax.axis_index("s")
    base_r = s * ROWS_OUT

    @pl.loop(0, K)
    def _(r):
        ones_v[r, :] = jnp.zeros((16,), jnp.float32)

    for t in range(ROWS_OUT // K):
        pltpu.sync_copy(ones_v, acc.at[pl.ds(base_r + t * K, K)])
    rem = ROWS_OUT % K
    if rem:
        pltpu.sync_copy(ones_v.at[pl.ds(0, rem)],
                        acc.at[pl.ds(base_r + (ROWS_OUT // K) * K, rem)])

    @pl.loop(0, K)
    def _(r):
        ones_v[r, :] = jnp.ones((16,), jnp.float32)

    plsc.subcore_barrier()

    base_e = c * (EN_PAD // NC) + s * PER_TILE

    @pl.loop(0, CHUNKS)
    def _(i):
        off = base_e + i * K
        pltpu.sync_copy(col_hbm.at[pl.ds(off, K)], cidx)
        pltpu.sync_copy(ones_v, acc.at[cidx], add=True)

    plsc.subcore_barrier()
    pltpu.sync_copy(acc.at[pl.ds(base_r, ROWS_OUT)],
                    out_hbm.at[c, pl.ds(base_r, ROWS_OUT)])


def _sc_deg(col_p):
    kern = pl.kernel(
        _sc_deg_body,
        out_type=jax.ShapeDtypeStruct((NC, NPAD, 16), jnp.float32),
        mesh=_MESH,
        scratch_types=[
            pltpu.VMEM_SHARED((NPAD, 16), jnp.float32),
            pltpu.VMEM((K, 16), jnp.float32),
            pltpu.VMEM((K,), jnp.int32),
        ],
    )
    return kern(col_p)


def _sc_agg_body(v_hbm, row_hbm, col_hbm, nq_hbm, out_hbm,
                 acc, rows_v, ridx, cidx, nqv):
    c = lax.axis_index("c")
    s = lax.axis_index("s")
    base_r = s * ROWS_OUT

    @pl.loop(0, K)
    def _(r):
        for g in range(8):
            rows_v[r, pl.ds(g * 16, 16)] = jnp.zeros((16,), jnp.float32)

    for t in range(ROWS_OUT // K):
        pltpu.sync_copy(rows_v, acc.at[pl.ds(base_r + t * K, K)])
    rem = ROWS_OUT % K
    if rem:
        pltpu.sync_copy(rows_v.at[pl.ds(0, rem)],
                        acc.at[pl.ds(base_r + (ROWS_OUT // K) * K, rem)])

    plsc.subcore_barrier()

    base_e = c * (EN_PAD // NC) + s * PER_TILE

    @pl.loop(0, CHUNKS)
    def _(i):
        off = base_e + i * K
        pltpu.sync_copy(row_hbm.at[pl.ds(off, K)], ridx)
        pltpu.sync_copy(col_hbm.at[pl.ds(off, K)], cidx)
        pltpu.sync_copy(nq_hbm.at[pl.ds(off, K)], nqv)
        pltpu.sync_copy(v_hbm.at[ridx], rows_v)

        @pl.loop(0, K)
        def _(k):
            lanes = jnp.zeros((16,), jnp.int32) + k
            nqk = plsc.load_gather(nqv, [lanes])
            for g in range(8):
                sl = pl.ds(g * 16, 16)
                rows_v[k, sl] = rows_v[k, sl] * nqk

        pltpu.sync_copy(rows_v, acc.at[cidx], add=True)

    plsc.subcore_barrier()
    pltpu.sync_copy(acc.at[pl.ds(base_r, ROWS_OUT)],
                    out_hbm.at[c, pl.ds(base_r, ROWS_OUT)])


def _sc_agg(v, row_p, col_p, nq_p):
    kern = pl.kernel(
        _sc_agg_body,
        out_type=jax.ShapeDtypeStruct((NC, NPAD, D_H), jnp.float32),
        mesh=_MESH,
        scratch_types=[
            pltpu.VMEM_SHARED((NPAD, D_H), jnp.float32),
            pltpu.VMEM((K, D_H), jnp.float32),
            pltpu.VMEM((K,), jnp.int32),
            pltpu.VMEM((K,), jnp.int32),
            pltpu.VMEM((K,), jnp.float32),
        ],
        compiler_params=_SC_PARAMS,
    )
    return kern(v, row_p, col_p, nq_p)


def _dis_from(deg_ref):
    d = deg_ref[0] + deg_ref[1]
    return lax.rsqrt(d[:, 0:1])


def _tc_layer1_body(x_ref, w_ref, deg_ref, o_ref):
    dis = _dis_from(deg_ref)
    o_ref[...] = jnp.dot(x_ref[...], w_ref[...],
                         preferred_element_type=jnp.float32) * dis


def _tc_layer1(x, W, degs):
    return pl.pallas_call(
        _tc_layer1_body,
        grid=(NBLK,),
        in_specs=[
            pl.BlockSpec((BLK, D_H), lambda i: (i, 0)),
            pl.BlockSpec((D_H, D_H), lambda i: (0, 0)),
            pl.BlockSpec((NC, BLK, 16), lambda i: (0, i, 0)),
        ],
        out_specs=pl.BlockSpec((BLK, D_H), lambda i: (i, 0)),
        out_shape=jax.ShapeDtypeStruct((N, D_H), jnp.float32),
    )(x, W, degs)


def _tc_layer_body(g_ref, deg_ref, b_ref, w_ref, o_ref):
    dis = _dis_from(deg_ref)
    h = jnp.maximum((g_ref[0] + g_ref[1]) * dis + b_ref[...], 0.0)
    o_ref[...] = jnp.dot(h, w_ref[...],
                         preferred_element_type=jnp.float32) * dis


def _tc_layer(g, degs, b, W):
    return pl.pallas_call(
        _tc_layer_body,
        grid=(NBLK,),
        in_specs=[
            pl.BlockSpec((NC, BLK, D_H), lambda i: (0, i, 0)),
            pl.BlockSpec((NC, BLK, 16), lambda i: (0, i, 0)),
            pl.BlockSpec((1, D_H), lambda i: (0, 0)),
            pl.BlockSpec((D_H, D_H), lambda i: (0, 0)),
        ],
        out_specs=pl.BlockSpec((BLK, D_H), lambda i: (i, 0)),
        out_shape=jax.ShapeDtypeStruct((N, D_H), jnp.float32),
    )(g, degs, b.reshape(1, D_H), W)


def _tc_final_body(g_ref, deg_ref, b_ref, batch_ref, wc_ref, bc_ref, o_ref,
                   sums_scr, cnt_scr):
    i = pl.program_id(0)

    @pl.when(i == 0)
    def _():
        sums_scr[...] = jnp.zeros_like(sums_scr)
        cnt_scr[...] = jnp.zeros_like(cnt_scr)

    dis = _dis_from(deg_ref)
    h = jnp.maximum((g_ref[0] + g_ref[1]) * dis + b_ref[...], 0.0)
    onehot = (batch_ref[...] ==
              lax.broadcasted_iota(jnp.int32, (1, G), 1)).astype(jnp.float32)
    sums_scr[...] += lax.dot_general(onehot, h, (((0,), (0,)), ((), ())),
                                     preferred_element_type=jnp.float32)
    cnt_scr[...] += lax.dot_general(onehot, jnp.ones((BLK, D_H), jnp.float32),
                                    (((0,), (0,)), ((), ())),
                                    preferred_element_type=jnp.float32)

    @pl.when(i == pl.num_programs(0) - 1)
    def _():
        pooled = sums_scr[...] / jnp.maximum(cnt_scr[...], 1.0)
        o_ref[...] = jnp.dot(pooled, wc_ref[...],
                             preferred_element_type=jnp.float32) + bc_ref[...]


def _tc_final(g, degs, b, batch2, Wc, bc):
    return pl.pallas_call(
        _tc_final_body,
        grid=(NBLK,),
        in_specs=[
            pl.BlockSpec((NC, BLK, D_H), lambda i: (0, i, 0)),
            pl.BlockSpec((NC, BLK, 16), lambda i: (0, i, 0)),
            pl.BlockSpec((1, D_H), lambda i: (0, 0)),
            pl.BlockSpec((BLK, 1), lambda i: (i, 0)),
            pl.BlockSpec((D_H, G), lambda i: (0, 0)),
            pl.BlockSpec((1, G), lambda i: (0, 0)),
        ],
        out_specs=pl.BlockSpec((G, G), lambda i: (0, 0)),
        out_shape=jax.ShapeDtypeStruct((G, G), jnp.float32),
        scratch_shapes=[
            pltpu.VMEM((G, D_H), jnp.float32),
            pltpu.VMEM((G, D_H), jnp.float32),
        ],
    )(g, degs, b.reshape(1, D_H), batch2, Wc, bc.reshape(1, G))


def kernel(x, edge_index, batch, node_centrality, edge_centrality,
           W1, b1, W2, b2, W3, b3, Wc, bc):
    loops = jnp.arange(N, dtype=jnp.int32)
    row = jnp.concatenate([edge_index[0], loops])
    col = jnp.concatenate([edge_index[1], loops])
    nq = jnp.concatenate([edge_centrality, node_centrality])

    pad = EN_PAD - EN
    row_p = jnp.concatenate([row, jnp.zeros((pad,), jnp.int32)])
    col_p = jnp.concatenate([col, jnp.full((pad,), N, jnp.int32)])
    nq_p = jnp.concatenate([nq, jnp.zeros((pad,), jnp.float32)])

    deg_p = _sc_deg(col_p)
    degs = deg_p[:, :N, :]

    v = _tc_layer1(x, W1, degs)
    g = _sc_agg(v, row_p, col_p, nq_p)
    v = _tc_layer(g[:, :N, :], degs, b1, W2)
    g = _sc_agg(v, row_p, col_p, nq_p)
    v = _tc_layer(g[:, :N, :], degs, b2, W3)
    g = _sc_agg(v, row_p, col_p, nq_p)

    batch2 = batch.reshape(N, 1)
    return _tc_final(g[:, :N, :], degs, b3, batch2, Wc, bc)


# speedup vs baseline: 10.7251x; 1.3236x over previous
import dataclasses

import jax
import jax.numpy as jnp
from jax import lax
from jax.experimental import pallas as pl
from jax.experimental.pallas import tpu as pltpu
from jax.experimental.pallas import tpu_sc as plsc

N = 10000
E = 320000
D_H = 128
G = 64

NC = 2
NS = 16
NW = NC * NS
K = 64
EN = E + N
CHUNKS = -(-EN // (NW * K))
PER_TILE = CHUNKS * K
EN_PAD = PER_TILE * NW
NPAD = ((N + 1 + NS * 8 - 1) // (NS * 8)) * NS * 8
ROWS_OUT = NPAD // NS
BLK = 1000
NBLK = N // BLK

_MESH = plsc.VectorSubcoreMesh(core_axis_name="c", subcore_axis_name="s",
                               num_cores=NC, num_subcores=NS)

_SC_PARAMS = pltpu.CompilerParams()
if "needs_layout_passes" in pltpu.CompilerParams.__dataclass_fields__:
    _SC_PARAMS = dataclasses.replace(_SC_PARAMS, needs_layout_passes=False)


def _sc_deg_body(meta_hbm, out_hbm, acc, ones_v, mv):
    c = lax.axis_index("c")
    s = lax.axis_index("s")
    base_r = s * ROWS_OUT

    @pl.loop(0, K)
    def _(r):
        ones_v[r, :] = jnp.zeros((16,), jnp.float32)

    for t in range(ROWS_OUT // K):
        pltpu.sync_copy(ones_v, acc.at[pl.ds(base_r + t * K, K)])
    rem = ROWS_OUT % K
    if rem:
        pltpu.sync_copy(ones_v.at[pl.ds(0, rem)],
                        acc.at[pl.ds(base_r + (ROWS_OUT // K) * K, rem)])

    @pl.loop(0, K)
    def _(r):
        ones_v[r, :] = jnp.ones((16,), jnp.float32)

    plsc.subcore_barrier()

    base_c = (c * NS + s) * CHUNKS

    @pl.loop(0, CHUNKS)
    def _(i):
        pltpu.sync_copy(meta_hbm.at[base_c + i], mv)
        pltpu.sync_copy(ones_v, acc.at[mv.at[1]], add=True)

    plsc.subcore_barrier()
    pltpu.sync_copy(acc.at[pl.ds(base_r, ROWS_OUT)],
                    out_hbm.at[c, pl.ds(base_r, ROWS_OUT)])


def _sc_deg(meta):
    kern = pl.kernel(
        _sc_deg_body,
        out_type=jax.ShapeDtypeStruct((NC, NPAD, 16), jnp.float32),
        mesh=_MESH,
        scratch_types=[
            pltpu.VMEM_SHARED((NPAD, 16), jnp.float32),
            pltpu.VMEM((K, 16), jnp.float32),
            pltpu.VMEM((2, K), jnp.int32),
        ],
        compiler_params=_SC_PARAMS,
    )
    return kern(meta)


def _sc_agg_body(v_hbm, meta_hbm, nq16_hbm, out_hbm,
                 acc, rows0, rows1, rows2, meta0, meta1, meta2,
                 nq0, nq1, nq2, gsem0, gsem1, gsem2, ssem0, ssem1, ssem2,
                 isem):
    rows = (rows0, rows1, rows2)
    meta = (meta0, meta1, meta2)
    nq = (nq0, nq1, nq2)
    gsem = (gsem0, gsem1, gsem2)
    ssem = (ssem0, ssem1, ssem2)

    c = lax.axis_index("c")
    s = lax.axis_index("s")
    base_r = s * ROWS_OUT

    @pl.loop(0, K)
    def _(r):
        for g in range(8):
            rows0[r, pl.ds(g * 16, 16)] = jnp.zeros((16,), jnp.float32)

    for t in range(ROWS_OUT // K):
        pltpu.sync_copy(rows0, acc.at[pl.ds(base_r + t * K, K)])
    rem = ROWS_OUT % K
    if rem:
        pltpu.sync_copy(rows0.at[pl.ds(0, rem)],
                        acc.at[pl.ds(base_r + (ROWS_OUT // K) * K, rem)])

    plsc.subcore_barrier()

    base_c = (c * NS + s) * CHUNKS

    def load_idx(j, b):
        d1 = pltpu.async_copy(meta_hbm.at[base_c + j], meta[b], isem)
        d2 = pltpu.async_copy(nq16_hbm.at[base_c + j], nq[b], isem)
        d1.wait()
        d2.wait()

    def start_gather(j, b):
        pltpu.async_copy(v_hbm.at[meta[b].at[0]], rows[b], gsem[b])

    def wait_gather(b):
        pltpu.make_async_copy(v_hbm.at[meta[b].at[0]], rows[b],
                              gsem[b]).wait()

    def scale(b):
        rb = rows[b]
        nb = nq[b]

        @plsc.parallel_loop(0, K, unroll=8)
        def _(k):
            nqk = nb[k, :]
            for g in range(8):
                sl = pl.ds(g * 16, 16)
                rb[k, sl] = rb[k, sl] * nqk

    def start_scatter(b):
        pltpu.async_copy(rows[b], acc.at[meta[b].at[1]], ssem[b], add=True)

    def wait_scatter(b):
        pltpu.make_async_copy(rows[b], acc.at[meta[b].at[1]],
                              ssem[b]).wait()

    load_idx(0, 0)
    start_gather(0, 0)
    for j in (0, 1):
        load_idx(j + 1, j + 1)
        start_gather(j + 1, j + 1)
        wait_gather(j)
        scale(j)
        start_scatter(j)

    @pl.loop(0, (CHUNKS - 3) // 3)
    def _(p):
        for u in range(3):
            j = 2 + 3 * p + u
            b = (2 + u) % 3
            wait_scatter(u)
            load_idx(j + 1, u)
            start_gather(j + 1, u)
            wait_gather(b)
            scale(b)
            start_scatter(b)

    jl = CHUNKS - 1
    bl = jl % 3
    wait_gather(bl)
    scale(bl)
    start_scatter(bl)
    for b in range(3):
        wait_scatter(b)

    plsc.subcore_barrier()
    pltpu.sync_copy(acc.at[pl.ds(base_r, ROWS_OUT)],
                    out_hbm.at[c, pl.ds(base_r, ROWS_OUT)])


def _sc_agg(v, meta, nq16):
    kern = pl.kernel(
        _sc_agg_body,
        out_type=jax.ShapeDtypeStruct((NC, NPAD, D_H), jnp.float32),
        mesh=_MESH,
        scratch_types=[
            pltpu.VMEM_SHARED((NPAD, D_H), jnp.float32),
            pltpu.VMEM((K, D_H), jnp.float32),
            pltpu.VMEM((K, D_H), jnp.float32),
            pltpu.VMEM((K, D_H), jnp.float32),
            pltpu.VMEM((2, K), jnp.int32),
            pltpu.VMEM((2, K), jnp.int32),
            pltpu.VMEM((2, K), jnp.int32),
            pltpu.VMEM((K, 16), jnp.float32),
            pltpu.VMEM((K, 16), jnp.float32),
            pltpu.VMEM((K, 16), jnp.float32),
            pltpu.SemaphoreType.DMA,
            pltpu.SemaphoreType.DMA,
            pltpu.SemaphoreType.DMA,
            pltpu.SemaphoreType.DMA,
            pltpu.SemaphoreType.DMA,
            pltpu.SemaphoreType.DMA,
            pltpu.SemaphoreType.DMA,
        ],
        compiler_params=_SC_PARAMS,
    )
    return kern(v, meta, nq16)


def _dis_from(deg_ref):
    d = deg_ref[0] + deg_ref[1]
    return lax.rsqrt(d[:, 0:1])


def _tc_layer1_body(x_ref, w_ref, deg_ref, o_ref):
    dis = _dis_from(deg_ref)
    o_ref[...] = jnp.dot(x_ref[...], w_ref[...],
                         preferred_element_type=jnp.float32) * dis


def _tc_layer1(x, W, degs):
    return pl.pallas_call(
        _tc_layer1_body,
        grid=(NBLK,),
        in_specs=[
            pl.BlockSpec((BLK, D_H), lambda i: (i, 0)),
            pl.BlockSpec((D_H, D_H), lambda i: (0, 0)),
            pl.BlockSpec((NC, BLK, 16), lambda i: (0, i, 0)),
        ],
        out_specs=pl.BlockSpec((BLK, D_H), lambda i: (i, 0)),
        out_shape=jax.ShapeDtypeStruct((N, D_H), jnp.float32),
    )(x, W, degs)


def _tc_layer_body(g_ref, deg_ref, b_ref, w_ref, o_ref):
    dis = _dis_from(deg_ref)
    h = jnp.maximum((g_ref[0] + g_ref[1]) * dis + b_ref[...], 0.0)
    o_ref[...] = jnp.dot(h, w_ref[...],
                         preferred_element_type=jnp.float32) * dis


def _tc_layer(g, degs, b, W):
    return pl.pallas_call(
        _tc_layer_body,
        grid=(NBLK,),
        in_specs=[
            pl.BlockSpec((NC, BLK, D_H), lambda i: (0, i, 0)),
            pl.BlockSpec((NC, BLK, 16), lambda i: (0, i, 0)),
            pl.BlockSpec((1, D_H), lambda i: (0, 0)),
            pl.BlockSpec((D_H, D_H), lambda i: (0, 0)),
        ],
        out_specs=pl.BlockSpec((BLK, D_H), lambda i: (i, 0)),
        out_shape=jax.ShapeDtypeStruct((N, D_H), jnp.float32),
    )(g, degs, b.reshape(1, D_H), W)


def _tc_final_body(g_ref, deg_ref, b_ref, batch_ref, wc_ref, bc_ref, o_ref,
                   sums_scr, cnt_scr):
    i = pl.program_id(0)

    @pl.when(i == 0)
    def _():
        sums_scr[...] = jnp.zeros_like(sums_scr)
        cnt_scr[...] = jnp.zeros_like(cnt_scr)

    dis = _dis_from(deg_ref)
    h = jnp.maximum((g_ref[0] + g_ref[1]) * dis + b_ref[...], 0.0)
    onehot = (batch_ref[...] ==
              lax.broadcasted_iota(jnp.int32, (1, G), 1)).astype(jnp.float32)
    sums_scr[...] += lax.dot_general(onehot, h, (((0,), (0,)), ((), ())),
                                     preferred_element_type=jnp.float32)
    cnt_scr[...] += lax.dot_general(onehot, jnp.ones((BLK, D_H), jnp.float32),
                                    (((0,), (0,)), ((), ())),
                                    preferred_element_type=jnp.float32)

    @pl.when(i == pl.num_programs(0) - 1)
    def _():
        pooled = sums_scr[...] / jnp.maximum(cnt_scr[...], 1.0)
        o_ref[...] = jnp.dot(pooled, wc_ref[...],
                             preferred_element_type=jnp.float32) + bc_ref[...]


def _tc_final(g, degs, b, batch2, Wc, bc):
    return pl.pallas_call(
        _tc_final_body,
        grid=(NBLK,),
        in_specs=[
            pl.BlockSpec((NC, BLK, D_H), lambda i: (0, i, 0)),
            pl.BlockSpec((NC, BLK, 16), lambda i: (0, i, 0)),
            pl.BlockSpec((1, D_H), lambda i: (0, 0)),
            pl.BlockSpec((BLK, 1), lambda i: (i, 0)),
            pl.BlockSpec((D_H, G), lambda i: (0, 0)),
            pl.BlockSpec((1, G), lambda i: (0, 0)),
        ],
        out_specs=pl.BlockSpec((G, G), lambda i: (0, 0)),
        out_shape=jax.ShapeDtypeStruct((G, G), jnp.float32),
        scratch_shapes=[
            pltpu.VMEM((G, D_H), jnp.float32),
            pltpu.VMEM((G, D_H), jnp.float32),
        ],
    )(g, degs, b.reshape(1, D_H), batch2, Wc, bc.reshape(1, G))


def kernel(x, edge_index, batch, node_centrality, edge_centrality,
           W1, b1, W2, b2, W3, b3, Wc, bc):
    loops = jnp.arange(N, dtype=jnp.int32)
    row = jnp.concatenate([edge_index[0], loops])
    col = jnp.concatenate([edge_index[1], loops])
    nq = jnp.concatenate([edge_centrality, node_centrality])

    pad = EN_PAD - EN
    row_p = jnp.concatenate([row, jnp.zeros((pad,), jnp.int32)])
    col_p = jnp.concatenate([col, jnp.full((pad,), N, jnp.int32)])
    nq_p = jnp.concatenate([nq, jnp.zeros((pad,), jnp.float32)])

    tot = EN_PAD // K
    meta = jnp.stack([row_p.reshape(tot, K), col_p.reshape(tot, K)], axis=1)
    nq16 = jnp.broadcast_to(nq_p[:, None], (EN_PAD, 16)).reshape(tot, K, 16)

    deg_p = _sc_deg(meta)
    degs = deg_p[:, :N, :]

    v = _tc_layer1(x, W1, degs)
    g = _sc_agg(v, meta, nq16)
    v = _tc_layer(g[:, :N, :], degs, b1, W2)
    g = _sc_agg(v, meta, nq16)
    v = _tc_layer(g[:, :N, :], degs, b2, W3)
    g = _sc_agg(v, meta, nq16)

    batch2 = batch.reshape(N, 1)
    return _tc_final(g[:, :N, :], degs, b3, batch2, Wc, bc)


# speedup vs baseline: 11.5596x; 1.0778x over previous
import dataclasses

import jax
import jax.numpy as jnp
from jax import lax
from jax.experimental import pallas as pl
from jax.experimental.pallas import tpu as pltpu
from jax.experimental.pallas import tpu_sc as plsc

N = 10000
E = 320000
D_H = 128
G = 64

NC = 2
NS = 16
NW = NC * NS
K = 64
EN = E + N
CHUNKS = -(-EN // (NW * K))
PER_TILE = CHUNKS * K
EN_PAD = PER_TILE * NW
NPAD = ((N + 1 + NS * 8 - 1) // (NS * 8)) * NS * 8
ROWS_OUT = NPAD // NS
BLK = 1000
NBLK = N // BLK

_MESH = plsc.VectorSubcoreMesh(core_axis_name="c", subcore_axis_name="s",
                               num_cores=NC, num_subcores=NS)

_SC_PARAMS = pltpu.CompilerParams()
if "needs_layout_passes" in pltpu.CompilerParams.__dataclass_fields__:
    _SC_PARAMS = dataclasses.replace(_SC_PARAMS, needs_layout_passes=False)


def _sc_agg_body(v_hbm, meta_hbm, nq_hbm, out_hbm,
                 acc, rows0, rows1, rows2, meta0, meta1, meta2, nq_all,
                 gsem0, gsem1, gsem2, ssem0, ssem1, ssem2, isem):
    rows = (rows0, rows1, rows2)
    meta = (meta0, meta1, meta2)
    gsem = (gsem0, gsem1, gsem2)
    ssem = (ssem0, ssem1, ssem2)

    c = lax.axis_index("c")
    s = lax.axis_index("s")
    base_r = s * ROWS_OUT
    wid = c * NS + s

    pltpu.sync_copy(nq_hbm.at[wid], nq_all)

    @pl.loop(0, K)
    def _(r):
        for g in range(8):
            rows0[r, pl.ds(g * 16, 16)] = jnp.zeros((16,), jnp.float32)

    for t in range(ROWS_OUT // K):
        pltpu.sync_copy(rows0, acc.at[pl.ds(base_r + t * K, K)])
    rem = ROWS_OUT % K
    if rem:
        pltpu.sync_copy(rows0.at[pl.ds(0, rem)],
                        acc.at[pl.ds(base_r + (ROWS_OUT // K) * K, rem)])

    plsc.subcore_barrier()

    def load_idx(j, b):
        pltpu.async_copy(meta_hbm.at[wid, j], meta[b], isem).wait()

    def start_gather(b):
        pltpu.async_copy(v_hbm.at[meta[b].at[0]], rows[b], gsem[b])

    def wait_gather(b):
        pltpu.make_async_copy(v_hbm.at[meta[b].at[0]], rows[b],
                              gsem[b]).wait()

    def scale(j, b):
        rb = rows[b]
        nrow = nq_all.at[j]

        @plsc.parallel_loop(0, K, unroll=8)
        def _(k):
            lanes = jnp.zeros((16,), jnp.int32) + k
            nqk = plsc.load_gather(nrow, [lanes])
            for g in range(8):
                sl = pl.ds(g * 16, 16)
                rb[k, sl] = rb[k, sl] * nqk

    def start_scatter(b):
        pltpu.async_copy(rows[b], acc.at[meta[b].at[1]], ssem[b],
                         add=True)

    def wait_scatter(b):
        pltpu.make_async_copy(rows[b], acc.at[meta[b].at[1]],
                              ssem[b]).wait()

    load_idx(0, 0)
    start_gather(0)
    for j in (0, 1):
        load_idx(j + 1, j + 1)
        start_gather(j + 1)
        wait_gather(j)
        scale(j, j)
        start_scatter(j)

    @pl.loop(0, (CHUNKS - 3) // 3)
    def _(p):
        for u in range(3):
            j = 2 + 3 * p + u
            b = (2 + u) % 3
            wait_scatter(u)
            load_idx(j + 1, u)
            start_gather(u)
            wait_gather(b)
            scale(j, b)
            start_scatter(b)

    jl = CHUNKS - 1
    bl = jl % 3
    wait_gather(bl)
    scale(jl, bl)
    start_scatter(bl)
    for b in range(3):
        wait_scatter(b)

    plsc.subcore_barrier()
    pltpu.sync_copy(acc.at[pl.ds(base_r, ROWS_OUT)],
                    out_hbm.at[c, pl.ds(base_r, ROWS_OUT)])


def _sc_agg(v, meta, nqc):
    kern = pl.kernel(
        _sc_agg_body,
        out_type=jax.ShapeDtypeStruct((NC, NPAD, D_H), jnp.float32),
        mesh=_MESH,
        scratch_types=[
            pltpu.VMEM_SHARED((NPAD, D_H), jnp.float32),
            pltpu.VMEM((K, D_H), jnp.float32),
            pltpu.VMEM((K, D_H), jnp.float32),
            pltpu.VMEM((K, D_H), jnp.float32),
            pltpu.VMEM((2, K), jnp.int32),
            pltpu.VMEM((2, K), jnp.int32),
            pltpu.VMEM((2, K), jnp.int32),
            pltpu.VMEM((CHUNKS, K), jnp.float32),
            pltpu.SemaphoreType.DMA,
            pltpu.SemaphoreType.DMA,
            pltpu.SemaphoreType.DMA,
            pltpu.SemaphoreType.DMA,
            pltpu.SemaphoreType.DMA,
            pltpu.SemaphoreType.DMA,
            pltpu.SemaphoreType.DMA,
        ],
        compiler_params=_SC_PARAMS,
    )
    return kern(v, meta, nqc)


def _dis_from(deg_ref):
    d = deg_ref[0] + deg_ref[1]
    return lax.rsqrt(d[:, 0:1])


def _tc_layer1_body(x_ref, w_ref, deg_ref, o_ref):
    dis = _dis_from(deg_ref)
    o_ref[...] = jnp.dot(x_ref[...], w_ref[...],
                         preferred_element_type=jnp.float32) * dis


def _tc_layer1(x, W, degs):
    return pl.pallas_call(
        _tc_layer1_body,
        grid=(NBLK,),
        in_specs=[
            pl.BlockSpec((BLK, D_H), lambda i: (i, 0)),
            pl.BlockSpec((D_H, D_H), lambda i: (0, 0)),
            pl.BlockSpec((NC, BLK, D_H), lambda i: (0, i, 0)),
        ],
        out_specs=pl.BlockSpec((BLK, D_H), lambda i: (i, 0)),
        out_shape=jax.ShapeDtypeStruct((N, D_H), jnp.float32),
    )(x, W, degs)


def _tc_layer_body(g_ref, deg_ref, b_ref, w_ref, o_ref):
    dis = _dis_from(deg_ref)
    gsum = g_ref[0] + g_ref[1]
    h = jnp.maximum(gsum * dis + b_ref[...], 0.0)
    o_ref[...] = jnp.dot(h, w_ref[...],
                         preferred_element_type=jnp.float32) * dis


def _tc_layer(g, degs, b, W):
    return pl.pallas_call(
        _tc_layer_body,
        grid=(NBLK,),
        in_specs=[
            pl.BlockSpec((NC, BLK, D_H), lambda i: (0, i, 0)),
            pl.BlockSpec((NC, BLK, D_H), lambda i: (0, i, 0)),
            pl.BlockSpec((1, D_H), lambda i: (0, 0)),
            pl.BlockSpec((D_H, D_H), lambda i: (0, 0)),
        ],
        out_specs=pl.BlockSpec((BLK, D_H), lambda i: (i, 0)),
        out_shape=jax.ShapeDtypeStruct((N, D_H), jnp.float32),
    )(g, degs, b.reshape(1, D_H), W)


def _tc_final_body(g_ref, deg_ref, b_ref, batch_ref, wc_ref, bc_ref, o_ref,
                   sums_scr, cnt_scr):
    i = pl.program_id(0)

    @pl.when(i == 0)
    def _():
        sums_scr[...] = jnp.zeros_like(sums_scr)
        cnt_scr[...] = jnp.zeros_like(cnt_scr)

    dis = _dis_from(deg_ref)
    gsum = g_ref[0] + g_ref[1]
    h = jnp.maximum(gsum * dis + b_ref[...], 0.0)
    onehot = (batch_ref[...] ==
              lax.broadcasted_iota(jnp.int32, (1, G), 1)).astype(jnp.float32)
    sums_scr[...] += lax.dot_general(onehot, h, (((0,), (0,)), ((), ())),
                                     preferred_element_type=jnp.float32)
    cnt_scr[...] += lax.dot_general(onehot, jnp.ones((BLK, D_H), jnp.float32),
                                    (((0,), (0,)), ((), ())),
                                    preferred_element_type=jnp.float32)

    @pl.when(i == pl.num_programs(0) - 1)
    def _():
        pooled = sums_scr[...] / jnp.maximum(cnt_scr[...], 1.0)
        o_ref[...] = jnp.dot(pooled, wc_ref[...],
                             preferred_element_type=jnp.float32) + bc_ref[...]


def _tc_final(g, degs, b, batch2, Wc, bc):
    return pl.pallas_call(
        _tc_final_body,
        grid=(NBLK,),
        in_specs=[
            pl.BlockSpec((NC, BLK, D_H), lambda i: (0, i, 0)),
            pl.BlockSpec((NC, BLK, D_H), lambda i: (0, i, 0)),
            pl.BlockSpec((1, D_H), lambda i: (0, 0)),
            pl.BlockSpec((BLK, 1), lambda i: (i, 0)),
            pl.BlockSpec((D_H, G), lambda i: (0, 0)),
            pl.BlockSpec((1, G), lambda i: (0, 0)),
        ],
        out_specs=pl.BlockSpec((G, G), lambda i: (0, 0)),
        out_shape=jax.ShapeDtypeStruct((G, G), jnp.float32),
        scratch_shapes=[
            pltpu.VMEM((G, D_H), jnp.float32),
            pltpu.VMEM((G, D_H), jnp.float32),
        ],
    )(g, degs, b.reshape(1, D_H), batch2, Wc, bc.reshape(1, G))


def kernel(x, edge_index, batch, node_centrality, edge_centrality,
           W1, b1, W2, b2, W3, b3, Wc, bc):
    loops = jnp.arange(N, dtype=jnp.int32)
    row = jnp.concatenate([edge_index[0], loops])
    col = jnp.concatenate([edge_index[1], loops])
    nq = jnp.concatenate([edge_centrality, node_centrality])

    pad = EN_PAD - EN
    row_p = jnp.concatenate([row, jnp.zeros((pad,), jnp.int32)])
    col_p = jnp.concatenate([col, jnp.full((pad,), N, jnp.int32)])
    nq_p = jnp.concatenate([nq, jnp.zeros((pad,), jnp.float32)])

    tot = EN_PAD // K
    meta = jnp.stack([row_p.reshape(tot, K), col_p.reshape(tot, K)],
                     axis=1).reshape(NW, CHUNKS, 2, K)
    nqc = nq_p.reshape(NW, CHUNKS, K)

    ind = jnp.concatenate([jnp.ones((EN,), jnp.float32),
                           jnp.zeros((EN_PAD - EN,), jnp.float32)])
    deg_p = _sc_agg(jnp.ones((N, D_H), jnp.float32), meta,
                    ind.reshape(NW, CHUNKS, K))
    degs = deg_p[:, :N, :]

    v = _tc_layer1(x, W1, degs)
    g = _sc_agg(v, meta, nqc)
    v = _tc_layer(g[:, :N, :], degs, b1, W2)
    g = _sc_agg(v, meta, nqc)
    v = _tc_layer(g[:, :N, :], degs, b2, W3)
    g = _sc_agg(v, meta, nqc)

    batch2 = batch.reshape(N, 1)
    return _tc_final(g[:, :N, :], degs, b3, batch2, Wc, bc)


# speedup vs baseline: 11.9021x; 1.0296x over previous
import dataclasses

import jax
import jax.numpy as jnp
from jax import lax
from jax.experimental import pallas as pl
from jax.experimental.pallas import tpu as pltpu
from jax.experimental.pallas import tpu_sc as plsc

N = 10000
E = 320000
D_H = 128
G = 64

NC = 2
NS = 16
NW = NC * NS
K = 96
EN = E + N
CHUNKS = -(-EN // (NW * K))
PER_TILE = CHUNKS * K
EN_PAD = PER_TILE * NW
NPAD = ((N + 1 + NS * 8 - 1) // (NS * 8)) * NS * 8
ROWS_OUT = NPAD // NS
BLK = 1000
NBLK = N // BLK

_MESH = plsc.VectorSubcoreMesh(core_axis_name="c", subcore_axis_name="s",
                               num_cores=NC, num_subcores=NS)

_SC_PARAMS = pltpu.CompilerParams()
if "needs_layout_passes" in pltpu.CompilerParams.__dataclass_fields__:
    _SC_PARAMS = dataclasses.replace(_SC_PARAMS, needs_layout_passes=False)


def _sc_agg_body(v_hbm, meta_hbm, nq_hbm, out_hbm,
                 acc, rows0, rows1, rows2, meta0, meta1, meta2,
                 nqb0, nqb1, nqb2,
                 gsem0, gsem1, gsem2, ssem0, ssem1, ssem2, isem):
    rows = (rows0, rows1, rows2)
    meta = (meta0, meta1, meta2)
    nqb = (nqb0, nqb1, nqb2)
    gsem = (gsem0, gsem1, gsem2)
    ssem = (ssem0, ssem1, ssem2)

    c = lax.axis_index("c")
    s = lax.axis_index("s")
    base_r = s * ROWS_OUT
    wid = c * NS + s

    @pl.loop(0, K)
    def _(r):
        for g in range(8):
            rows0[r, pl.ds(g * 16, 16)] = jnp.zeros((16,), jnp.float32)

    for t in range(ROWS_OUT // K):
        pltpu.sync_copy(rows0, acc.at[pl.ds(base_r + t * K, K)])
    rem = ROWS_OUT % K
    if rem:
        pltpu.sync_copy(rows0.at[pl.ds(0, rem)],
                        acc.at[pl.ds(base_r + (ROWS_OUT // K) * K, rem)])

    plsc.subcore_barrier()

    def load_idx(j, b):
        d1 = pltpu.async_copy(meta_hbm.at[wid, j], meta[b], isem)
        d2 = pltpu.async_copy(nq_hbm.at[wid, j], nqb[b], isem)
        d1.wait()
        d2.wait()

    def start_gather(b):
        pltpu.async_copy(v_hbm.at[meta[b].at[0]], rows[b], gsem[b])

    def wait_gather(b):
        pltpu.make_async_copy(v_hbm.at[meta[b].at[0]], rows[b],
                              gsem[b]).wait()

    def scale(j, b):
        rb = rows[b]
        nrow = nqb[b]

        @plsc.parallel_loop(0, K, unroll=8)
        def _(k):
            lanes = jnp.zeros((16,), jnp.int32) + k
            nqk = plsc.load_gather(nrow, [lanes])
            for g in range(8):
                sl = pl.ds(g * 16, 16)
                rb[k, sl] = rb[k, sl] * nqk

    def start_scatter(b):
        pltpu.async_copy(rows[b], acc.at[meta[b].at[1]], ssem[b],
                         add=True)

    def wait_scatter(b):
        pltpu.make_async_copy(rows[b], acc.at[meta[b].at[1]],
                              ssem[b]).wait()

    load_idx(0, 0)
    start_gather(0)
    for j in (0, 1):
        load_idx(j + 1, j + 1)
        start_gather(j + 1)
        wait_gather(j)
        scale(j, j)
        start_scatter(j)

    @pl.loop(0, (CHUNKS - 3) // 3)
    def _(p):
        for u in range(3):
            j = 2 + 3 * p + u
            b = (2 + u) % 3
            wait_scatter(u)
            load_idx(j + 1, u)
            start_gather(u)
            wait_gather(b)
            scale(j, b)
            start_scatter(b)

    jl = CHUNKS - 1
    bl = jl % 3
    wait_gather(bl)
    scale(jl, bl)
    start_scatter(bl)
    for b in range(3):
        wait_scatter(b)

    plsc.subcore_barrier()
    pltpu.sync_copy(acc.at[pl.ds(base_r, ROWS_OUT)],
                    out_hbm.at[c, pl.ds(base_r, ROWS_OUT)])


def _sc_agg(v, meta, nqc):
    kern = pl.kernel(
        _sc_agg_body,
        out_type=jax.ShapeDtypeStruct((NC, NPAD, D_H), jnp.float32),
        mesh=_MESH,
        scratch_types=[
            pltpu.VMEM_SHARED((NPAD, D_H), jnp.float32),
            pltpu.VMEM((K, D_H), jnp.float32),
            pltpu.VMEM((K, D_H), jnp.float32),
            pltpu.VMEM((K, D_H), jnp.float32),
            pltpu.VMEM((2, K), jnp.int32),
            pltpu.VMEM((2, K), jnp.int32),
            pltpu.VMEM((2, K), jnp.int32),
            pltpu.VMEM((K,), jnp.float32),
            pltpu.VMEM((K,), jnp.float32),
            pltpu.VMEM((K,), jnp.float32),
            pltpu.SemaphoreType.DMA,
            pltpu.SemaphoreType.DMA,
            pltpu.SemaphoreType.DMA,
            pltpu.SemaphoreType.DMA,
            pltpu.SemaphoreType.DMA,
            pltpu.SemaphoreType.DMA,
            pltpu.SemaphoreType.DMA,
        ],
        compiler_params=_SC_PARAMS,
    )
    return kern(v, meta, nqc)


def _dis_from(deg_ref):
    d = deg_ref[0] + deg_ref[1]
    return lax.rsqrt(d[:, 0:1])


def _tc_layer1_body(x_ref, w_ref, deg_ref, o_ref):
    dis = _dis_from(deg_ref)
    o_ref[...] = jnp.dot(x_ref[...], w_ref[...],
                         preferred_element_type=jnp.float32) * dis


def _tc_layer1(x, W, degs):
    return pl.pallas_call(
        _tc_layer1_body,
        grid=(NBLK,),
        in_specs=[
            pl.BlockSpec((BLK, D_H), lambda i: (i, 0)),
            pl.BlockSpec((D_H, D_H), lambda i: (0, 0)),
            pl.BlockSpec((NC, BLK, D_H), lambda i: (0, i, 0)),
        ],
        out_specs=pl.BlockSpec((BLK, D_H), lambda i: (i, 0)),
        out_shape=jax.ShapeDtypeStruct((N, D_H), jnp.float32),
    )(x, W, degs)


def _tc_layer_body(g_ref, deg_ref, b_ref, w_ref, o_ref):
    dis = _dis_from(deg_ref)
    gsum = g_ref[0] + g_ref[1]
    h = jnp.maximum(gsum * dis + b_ref[...], 0.0)
    o_ref[...] = jnp.dot(h, w_ref[...],
                         preferred_element_type=jnp.float32) * dis


def _tc_layer(g, degs, b, W):
    return pl.pallas_call(
        _tc_layer_body,
        grid=(NBLK,),
        in_specs=[
            pl.BlockSpec((NC, BLK, D_H), lambda i: (0, i, 0)),
            pl.BlockSpec((NC, BLK, D_H), lambda i: (0, i, 0)),
            pl.BlockSpec((1, D_H), lambda i: (0, 0)),
            pl.BlockSpec((D_H, D_H), lambda i: (0, 0)),
        ],
        out_specs=pl.BlockSpec((BLK, D_H), lambda i: (i, 0)),
        out_shape=jax.ShapeDtypeStruct((N, D_H), jnp.float32),
    )(g, degs, b.reshape(1, D_H), W)


def _tc_final_body(g_ref, deg_ref, b_ref, batch_ref, wc_ref, bc_ref, o_ref,
                   sums_scr, cnt_scr):
    i = pl.program_id(0)

    @pl.when(i == 0)
    def _():
        sums_scr[...] = jnp.zeros_like(sums_scr)
        cnt_scr[...] = jnp.zeros_like(cnt_scr)

    dis = _dis_from(deg_ref)
    gsum = g_ref[0] + g_ref[1]
    h = jnp.maximum(gsum * dis + b_ref[...], 0.0)
    onehot = (batch_ref[...] ==
              lax.broadcasted_iota(jnp.int32, (1, G), 1)).astype(jnp.float32)
    sums_scr[...] += lax.dot_general(onehot, h, (((0,), (0,)), ((), ())),
                                     preferred_element_type=jnp.float32)
    cnt_scr[...] += lax.dot_general(onehot, jnp.ones((BLK, D_H), jnp.float32),
                                    (((0,), (0,)), ((), ())),
                                    preferred_element_type=jnp.float32)

    @pl.when(i == pl.num_programs(0) - 1)
    def _():
        pooled = sums_scr[...] / jnp.maximum(cnt_scr[...], 1.0)
        o_ref[...] = jnp.dot(pooled, wc_ref[...],
                             preferred_element_type=jnp.float32) + bc_ref[...]


def _tc_final(g, degs, b, batch2, Wc, bc):
    return pl.pallas_call(
        _tc_final_body,
        grid=(NBLK,),
        in_specs=[
            pl.BlockSpec((NC, BLK, D_H), lambda i: (0, i, 0)),
            pl.BlockSpec((NC, BLK, D_H), lambda i: (0, i, 0)),
            pl.BlockSpec((1, D_H), lambda i: (0, 0)),
            pl.BlockSpec((BLK, 1), lambda i: (i, 0)),
            pl.BlockSpec((D_H, G), lambda i: (0, 0)),
            pl.BlockSpec((1, G), lambda i: (0, 0)),
        ],
        out_specs=pl.BlockSpec((G, G), lambda i: (0, 0)),
        out_shape=jax.ShapeDtypeStruct((G, G), jnp.float32),
        scratch_shapes=[
            pltpu.VMEM((G, D_H), jnp.float32),
            pltpu.VMEM((G, D_H), jnp.float32),
        ],
    )(g, degs, b.reshape(1, D_H), batch2, Wc, bc.reshape(1, G))


def kernel(x, edge_index, batch, node_centrality, edge_centrality,
           W1, b1, W2, b2, W3, b3, Wc, bc):
    loops = jnp.arange(N, dtype=jnp.int32)
    row = jnp.concatenate([edge_index[0], loops])
    col = jnp.concatenate([edge_index[1], loops])
    nq = jnp.concatenate([edge_centrality, node_centrality])

    pad = EN_PAD - EN
    row_p = jnp.concatenate([row, jnp.zeros((pad,), jnp.int32)])
    col_p = jnp.concatenate([col, jnp.full((pad,), N, jnp.int32)])
    nq_p = jnp.concatenate([nq, jnp.zeros((pad,), jnp.float32)])

    tot = EN_PAD // K
    meta = jnp.stack([row_p.reshape(tot, K), col_p.reshape(tot, K)],
                     axis=1).reshape(NW, CHUNKS, 2, K)
    nqc = nq_p.reshape(NW, CHUNKS, K)

    ind = jnp.concatenate([jnp.ones((EN,), jnp.float32),
                           jnp.zeros((EN_PAD - EN,), jnp.float32)])
    deg_p = _sc_agg(jnp.ones((N, D_H), jnp.float32), meta,
                    ind.reshape(NW, CHUNKS, K))
    degs = deg_p[:, :N, :]

    v = _tc_layer1(x, W1, degs)
    g = _sc_agg(v, meta, nqc)
    v = _tc_layer(g[:, :N, :], degs, b1, W2)
    g = _sc_agg(v, meta, nqc)
    v = _tc_layer(g[:, :N, :], degs, b2, W3)
    g = _sc_agg(v, meta, nqc)

    batch2 = batch.reshape(N, 1)
    return _tc_final(g[:, :N, :], degs, b3, batch2, Wc, bc)
